# Initial kernel scaffold; baseline (speedup 1.0000x reference)
import jax
import jax.numpy as jnp
from jax.experimental import pallas as pl


def kernel(x, blocks0, blocks1, pos_edge_index, neg_edge_index, W1, b1, W2, b2):
    raise NotImplementedError("write your pallas kernel here")


# speedup vs baseline: 1.8637x; 1.8637x over previous
import functools

import jax
import jax.numpy as jnp
from jax import lax
from jax.experimental import pallas as pl
from jax.experimental.pallas import tpu as pltpu
from jax.experimental.pallas import tpu_sc as plsc

NN = 10000
NE = 320000
D = 128
NC, NS = 2, 16
NW = NC * NS
HS = 10240
NHIST = 4 * HS
CH = 128
DEG_CHUNKS = (4 * NE) // CH
AGG_CHUNKS = NE // CH
SCORE_CHUNKS = (2 * NE) // CH
NNP = 10240
ROWS_PER_TILE = NNP // NS
HIST_PER_TILE = NHIST // NS

def _degrees_body(idxf_hbm, zeros_hbm, ones_hbm, out_hbm, hist, idxb, ones_v):
    c = lax.axis_index("c")
    s = lax.axis_index("s")
    w = s * NC + c
    pltpu.sync_copy(zeros_hbm, hist.at[pl.ds(s * HIST_PER_TILE, HIST_PER_TILE)])
    pltpu.sync_copy(ones_hbm, ones_v)
    plsc.subcore_barrier()

    def chunk(j, carry):
        cidx = j * NW + w

        @pl.when(cidx < DEG_CHUNKS)
        def _():
            pltpu.sync_copy(idxf_hbm.at[pl.ds(cidx * CH, CH)], idxb.at[0])
            pltpu.sync_copy(ones_v, hist.at[idxb.at[0]], add=True)

        return carry

    lax.fori_loop(0, (DEG_CHUNKS + NW - 1) // NW, chunk, 0)
    plsc.subcore_barrier()
    sl = pl.ds(s * HIST_PER_TILE, HIST_PER_TILE)
    pltpu.sync_copy(hist.at[sl], out_hbm.at[c, 0, sl])


def _agg_body(src_hbm, dst_hbm, z_hbm, zrows_hbm, out_hbm, acc, sidx, didx, rows, sem):
    c = lax.axis_index("c")
    s = lax.axis_index("s")
    w = s * NC + c
    pltpu.sync_copy(zrows_hbm, acc.at[pl.ds(s * ROWS_PER_TILE, ROWS_PER_TILE)])
    plsc.subcore_barrier()

    def chunk(j, carry):
        cidx = j * NW + w

        @pl.when(cidx < AGG_CHUNKS)
        def _():
            base = cidx * CH
            pltpu.sync_copy(src_hbm.at[pl.ds(base, CH)], sidx)
            pltpu.sync_copy(dst_hbm.at[pl.ds(base, CH)], didx.at[0])
            pltpu.async_copy(z_hbm.at[sidx], rows, sem).wait()
            pltpu.sync_copy(rows, acc.at[didx.at[0]], add=True)

        return carry

    lax.fori_loop(0, (AGG_CHUNKS + NW - 1) // NW, chunk, 0)
    plsc.subcore_barrier()
    sl = pl.ds(s * ROWS_PER_TILE, ROWS_PER_TILE)
    pltpu.sync_copy(acc.at[sl], out_hbm.at[c, sl])


def _score_body(esrc_hbm, edst_hbm, h_hbm, out_hbm, sidx, didx, srows, drows, obuf, sem1, sem2):
    c = lax.axis_index("c")
    s = lax.axis_index("s")
    w = s * NC + c

    def chunk(j, carry):
        cidx = j * NW + w

        @pl.when(cidx < SCORE_CHUNKS)
        def _():
            base = cidx * CH
            pltpu.sync_copy(esrc_hbm.at[pl.ds(base, CH)], sidx)
            pltpu.sync_copy(edst_hbm.at[pl.ds(base, CH)], didx)
            cp1 = pltpu.async_copy(h_hbm.at[sidx], srows, sem1)
            cp2 = pltpu.async_copy(h_hbm.at[didx], drows, sem2)
            cp1.wait()
            cp2.wait()
            for g in range(CH // 16):
                eids = lax.iota(jnp.int32, 16) + (g * 16)

                def dot_step(k, acc):
                    kv = jnp.zeros((16,), jnp.int32) + k
                    a = plsc.load_gather(srows, [eids, kv])
                    b = plsc.load_gather(drows, [eids, kv])
                    return acc + a * b

                accv = lax.fori_loop(0, D, dot_step, jnp.zeros((16,), jnp.float32))
                obuf[pl.ds(g * 16, 16)] = accv
            pltpu.sync_copy(obuf, out_hbm.at[pl.ds(base, CH)])

        return carry

    lax.fori_loop(0, (SCORE_CHUNKS + NW - 1) // NW, chunk, 0)


@functools.lru_cache(maxsize=None)
def _sc_kernels():
    mesh = plsc.VectorSubcoreMesh(
        core_axis_name="c", subcore_axis_name="s",
        num_cores=NC, num_subcores=NS)
    degrees = pl.kernel(
        _degrees_body,
        out_type=jax.ShapeDtypeStruct((NC, 1, NHIST), jnp.float32),
        mesh=mesh,
        scratch_types=[
            pltpu.VMEM_SHARED((NHIST,), jnp.float32),
            pltpu.VMEM((1, CH), jnp.int32),
            pltpu.VMEM((CH,), jnp.float32),
        ],
    )
    agg = pl.kernel(
        _agg_body,
        out_type=jax.ShapeDtypeStruct((NC, NNP, D), jnp.float32),
        mesh=mesh,
        scratch_types=[
            pltpu.VMEM_SHARED((NNP, D), jnp.float32),
            pltpu.VMEM((CH,), jnp.int32),
            pltpu.VMEM((1, CH), jnp.int32),
            pltpu.VMEM((CH, D), jnp.float32),
            pltpu.SemaphoreType.DMA,
        ],
    )
    score = pl.kernel(
        _score_body,
        out_type=jax.ShapeDtypeStruct((2 * NE,), jnp.float32),
        mesh=mesh,
        scratch_types=[
            pltpu.VMEM((CH,), jnp.int32),
            pltpu.VMEM((CH,), jnp.int32),
            pltpu.VMEM((CH, D), jnp.float32),
            pltpu.VMEM((CH, D), jnp.float32),
            pltpu.VMEM((CH,), jnp.float32),
            pltpu.SemaphoreType.DMA,
            pltpu.SemaphoreType.DMA,
        ],
        compiler_params=pltpu.CompilerParams(needs_layout_passes=False),
    )
    return degrees, agg, score


def _norm(deg):
    return jnp.where(deg > 0, lax.rsqrt(jnp.maximum(deg, 1.0)), 0.0)


def _tc_pre_body(degp_ref, x_ref, w_ref, z_ref):
    dp = degp_ref[...]
    d = dp[0, 0] + dp[1, 0]
    n_out = _norm(lax.slice(d, (0,), (NN,)))
    xs = x_ref[...] * n_out[:, None]
    z_ref[...] = jnp.dot(xs, w_ref[...], preferred_element_type=jnp.float32)


_tc_pre = pl.pallas_call(
    _tc_pre_body, out_shape=jax.ShapeDtypeStruct((NN, D), jnp.float32))


def _tc_mid_body(degp_ref, aggp_ref, b1_ref, w2_ref, z_ref):
    dp = degp_ref[...]
    d = dp[0, 0] + dp[1, 0]
    n_in0 = _norm(lax.slice(d, (HS,), (HS + NN,)))
    n_out1 = _norm(lax.slice(d, (2 * HS,), (2 * HS + NN,)))
    aggp = aggp_ref[...]
    agg = aggp[0, :NN] + aggp[1, :NN]
    h = jnp.maximum(agg * n_in0[:, None] + b1_ref[...][None, :], 0.0)
    z_ref[...] = jnp.dot(h * n_out1[:, None], w2_ref[...],
                         preferred_element_type=jnp.float32)


_tc_mid = pl.pallas_call(
    _tc_mid_body, out_shape=jax.ShapeDtypeStruct((NN, D), jnp.float32))


def _tc_post_body(degp_ref, aggp_ref, b2_ref, h_ref):
    dp = degp_ref[...]
    d = dp[0, 0] + dp[1, 0]
    n_in1 = _norm(lax.slice(d, (3 * HS,), (3 * HS + NN,)))
    aggp = aggp_ref[...]
    agg = aggp[0, :NN] + aggp[1, :NN]
    h_ref[...] = jnp.maximum(agg * n_in1[:, None] + b2_ref[...][None, :], 0.0)


_tc_post = pl.pallas_call(
    _tc_post_body, out_shape=jax.ShapeDtypeStruct((NN, D), jnp.float32))


def kernel(x, blocks0, blocks1, pos_edge_index, neg_edge_index, W1, b1, W2, b2):
    src0 = blocks0[0].astype(jnp.int32)
    dst0 = blocks0[1].astype(jnp.int32)
    src1 = blocks1[0].astype(jnp.int32)
    dst1 = blocks1[1].astype(jnp.int32)
    idxf = jnp.concatenate([src0, dst0 + HS, src1 + 2 * HS, dst1 + 3 * HS])
    zeros_h = jnp.zeros((HIST_PER_TILE,), jnp.float32)
    ones_h = jnp.ones((CH,), jnp.float32)
    zrows = jnp.zeros((ROWS_PER_TILE, D), jnp.float32)

    degrees_k, agg_k, score_k = _sc_kernels()
    degp = degrees_k(idxf, zeros_h, ones_h)
    z1 = _tc_pre(degp, x, W1)
    aggp1 = agg_k(src0, dst0, z1, zrows)
    z2 = _tc_mid(degp, aggp1, b1, W2)
    aggp2 = agg_k(src1, dst1, z2, zrows)
    h2 = _tc_post(degp, aggp2, b2)

    esrc = jnp.concatenate([pos_edge_index[0], neg_edge_index[0]]).astype(jnp.int32)
    edst = jnp.concatenate([pos_edge_index[1], neg_edge_index[1]]).astype(jnp.int32)
    scores = score_k(esrc, edst, h2)
    return (scores[:NE, None], scores[NE:, None])
